# baseline (speedup 1.0000x reference)
import functools

import jax
import jax.numpy as jnp
from jax import lax
from jax.experimental import pallas as pl
from jax.experimental.pallas import tpu as pltpu
from jax.experimental.pallas import tpu_sc as plsc

N = 10000
E = 320000
D_IN = 128
D_HID = 256
NC = 2
NS = 16
LANES = 16
CHUNK = 80
ROWS_PER_TILE = N // NS
NCHUNKS_TOT = E // CHUNK
PAD_CHUNKS = 4002


RING1 = 3
GDEPTH1 = 2
U1 = 9
UNROLL1 = 2 * U1


def _make_sc_agg_l1():
    chunks_per_tile = NCHUNKS_TOT // (NC * NS)
    K = chunks_per_tile
    mesh = plsc.VectorSubcoreMesh(core_axis_name="c", subcore_axis_name="s")

    scratch = [pltpu.VMEM((2, U1, 2, CHUNK), jnp.int32)]
    scratch += [pltpu.VMEM((CHUNK, D_IN), jnp.float32) for _ in range(RING1)]
    scratch += [
        pltpu.VMEM((CHUNK, LANES), jnp.float32),
        pltpu.VMEM_SHARED((N, D_IN), jnp.float32),
        pltpu.VMEM_SHARED((N, LANES), jnp.float32),
    ]
    scratch += [pltpu.SemaphoreType.DMA for _ in range(3 * RING1 + 2)]

    @functools.partial(
        pl.kernel,
        out_type=(jax.ShapeDtypeStruct((NC * N, D_IN), jnp.float32),
                  jax.ShapeDtypeStruct((NC * N, LANES), jnp.float32)),
        mesh=mesh,
        scratch_types=scratch,
        compiler_params=pltpu.CompilerParams(use_tc_tiling_on_sc=False),
    )
    def agg(table_hbm, edges_hbm, z_hbm, z16_hbm, out_hbm, outc_hbm,
            ebuf, *rest):
        rows = list(rest[:RING1])
        ones = rest[RING1]
        acc = rest[RING1 + 1]
        accc = rest[RING1 + 2]
        gsems = list(rest[RING1 + 3:RING1 + 3 + RING1])
        ssems = list(rest[RING1 + 3 + RING1:RING1 + 3 + 2 * RING1])
        csems = list(rest[RING1 + 3 + 2 * RING1:RING1 + 3 + 3 * RING1])
        isems = list(rest[RING1 + 3 + 3 * RING1:])

        c = lax.axis_index("c")
        s = lax.axis_index("s")
        rbase = s * ROWS_PER_TILE
        cbase = (c * NS + s) * chunks_per_tile

        def load_group(gi, p):
            pltpu.async_copy(edges_hbm.at[pl.ds(cbase + gi * U1, U1)],
                             ebuf.at[p], isems[p])

        def wait_group(p):
            pltpu.make_async_copy(edges_hbm.at[pl.ds(0, U1)],
                                  ebuf.at[p], isems[p]).wait()

        def drain_gather(b):
            pltpu.make_async_copy(table_hbm.at[pl.ds(0, CHUNK)],
                                  rows[b], gsems[b]).wait()

        def drain_scatters(b):
            pltpu.make_async_copy(z_hbm.at[pl.ds(0, CHUNK)],
                                  rows[b], ssems[b]).wait()
            pltpu.make_async_copy(z16_hbm.at[pl.ds(0, CHUNK)],
                                  ones, csems[b]).wait()

        pltpu.sync_copy(edges_hbm.at[pl.ds(cbase, U1)], ebuf.at[0])
        for j0 in range(GDEPTH1):
            pltpu.async_copy(table_hbm.at[ebuf.at[0, j0, 0]],
                             rows[j0], gsems[j0])

        def fill(i, carry):
            ones[i] = jnp.full((LANES,), 1.0, jnp.float32)
            return carry
        lax.fori_loop(0, CHUNK, fill, 0)
        pltpu.sync_copy(z16_hbm.at[pl.ds(0, ROWS_PER_TILE)],
                        accc.at[pl.ds(rbase, ROWS_PER_TILE)])
        pltpu.sync_copy(z_hbm.at[pl.ds(0, ROWS_PER_TILE)],
                        acc.at[pl.ds(rbase, ROWS_PER_TILE)])
        plsc.subcore_barrier()

        def body(g, carry):
            base = g * UNROLL1
            for u in range(UNROLL1):
                j = base + u
                rs = u % RING1
                p = u // U1
                ei = u % U1

                if u == 1:
                    @pl.when((2 * g + 1) * U1 < K)
                    def _load_b():
                        load_group(2 * g + 1, 1)
                if u == U1 + 1:
                    @pl.when((2 * g + 2) * U1 < K)
                    def _load_a():
                        load_group(2 * g + 2, 0)
                if u == U1 - GDEPTH1:
                    @pl.when((2 * g + 1) * U1 < K)
                    def _wait_b():
                        wait_group(1)
                if u == UNROLL1 - GDEPTH1:
                    @pl.when((2 * g + 2) * U1 < K)
                    def _wait_a():
                        wait_group(0)

                @pl.when(j < K)
                def _process():
                    drain_gather(rs)
                    pltpu.async_copy(rows[rs], acc.at[ebuf.at[p, ei, 1]],
                                     ssems[rs], add=True)
                    pltpu.async_copy(ones, accc.at[ebuf.at[p, ei, 1]],
                                     csems[rs], add=True)
                    jj = j + GDEPTH1
                    pj = ((u + GDEPTH1) // U1) % 2
                    ej = (u + GDEPTH1) % U1
                    bg = (u + GDEPTH1) % RING1

                    @pl.when(jj < K)
                    def _prefetch():
                        @pl.when(jj >= RING1)
                        def _drain_prev():
                            drain_scatters(bg)
                        pltpu.async_copy(table_hbm.at[ebuf.at[pj, ej, 0]],
                                         rows[bg], gsems[bg])
            return carry

        lax.fori_loop(0, (K + UNROLL1 - 1) // UNROLL1, body, 0)
        for b in range(RING1):
            drain_scatters(b)
        plsc.subcore_barrier()
        pltpu.sync_copy(acc.at[pl.ds(rbase, ROWS_PER_TILE)],
                        out_hbm.at[pl.ds(c * N + rbase, ROWS_PER_TILE)])
        pltpu.sync_copy(accc.at[pl.ds(rbase, ROWS_PER_TILE)],
                        outc_hbm.at[pl.ds(c * N + rbase, ROWS_PER_TILE)])

    return agg


RING = 4
GDEPTH = 2
U2 = 12
UNROLL2 = 2 * U2


def _make_sc_agg_l2():
    chunks_per_tile = NCHUNKS_TOT // NS
    mesh = plsc.VectorSubcoreMesh(core_axis_name="c", subcore_axis_name="s")
    K = chunks_per_tile

    scratch = [pltpu.VMEM((2, U2, 2, CHUNK), jnp.int32)]
    scratch += [pltpu.VMEM((CHUNK, D_IN), jnp.float32) for _ in range(RING)]
    scratch += [pltpu.VMEM_SHARED((N, D_IN), jnp.float32)]
    scratch += [pltpu.SemaphoreType.DMA for _ in range(2 * RING + 2)]

    @functools.partial(
        pl.kernel,
        out_type=jax.ShapeDtypeStruct((NC * N, D_IN), jnp.float32),
        mesh=mesh,
        scratch_types=scratch,
        compiler_params=pltpu.CompilerParams(use_tc_tiling_on_sc=False),
    )
    def agg(table_hbm, edges_hbm, edgesN_hbm, z_hbm, out_hbm, ebuf, *rest):
        rows = list(rest[:RING])
        acc = rest[RING]
        gsems = list(rest[RING + 1:RING + 1 + RING])
        ssems = list(rest[RING + 1 + RING:RING + 1 + 2 * RING])
        isems = list(rest[RING + 1 + 2 * RING:])

        c = lax.axis_index("c")
        s = lax.axis_index("s")
        rbase = s * ROWS_PER_TILE
        cbase = s * chunks_per_tile

        def load_group(gi, p):
            @pl.when(c == 0)
            def _load0():
                pltpu.async_copy(edges_hbm.at[pl.ds(cbase + gi * U2, U2)],
                                 ebuf.at[p], isems[p])

            @pl.when(c == 1)
            def _load1():
                pltpu.async_copy(edgesN_hbm.at[pl.ds(cbase + gi * U2, U2)],
                                 ebuf.at[p], isems[p])

        def wait_group(p):
            pltpu.make_async_copy(edges_hbm.at[pl.ds(0, U2)],
                                  ebuf.at[p], isems[p]).wait()

        def drain_gather(b):
            pltpu.make_async_copy(table_hbm.at[pl.ds(0, CHUNK)],
                                  rows[b], gsems[b]).wait()

        def drain_scatter(b):
            pltpu.make_async_copy(z_hbm.at[pl.ds(0, CHUNK)],
                                  rows[b], ssems[b]).wait()

        load_group(0, 0)
        wait_group(0)
        for j0 in range(GDEPTH):
            pltpu.async_copy(table_hbm.at[ebuf.at[0, j0, 0]],
                             rows[j0], gsems[j0])
        pltpu.sync_copy(z_hbm.at[pl.ds(0, ROWS_PER_TILE)],
                        acc.at[pl.ds(rbase, ROWS_PER_TILE)])
        plsc.subcore_barrier()

        def body(g, carry):
            base = g * UNROLL2
            for u in range(UNROLL2):
                j = base + u
                rs = u % RING
                p = u // U2
                ei = u % U2

                if u == 2:
                    @pl.when((2 * g + 1) * U2 < K)
                    def _load_b():
                        load_group(2 * g + 1, 1)
                if u == U2 + 2:
                    @pl.when((2 * g + 2) * U2 < K)
                    def _load_a():
                        load_group(2 * g + 2, 0)
                if u == U2 - GDEPTH:
                    @pl.when((2 * g + 1) * U2 < K)
                    def _wait_b():
                        wait_group(1)
                if u == UNROLL2 - GDEPTH:
                    @pl.when((2 * g + 2) * U2 < K)
                    def _wait_a():
                        wait_group(0)

                @pl.when(j < K)
                def _process():
                    drain_gather(rs)
                    pltpu.async_copy(rows[rs], acc.at[ebuf.at[p, ei, 1]],
                                     ssems[rs], add=True)
                    jj = j + GDEPTH
                    pj = ((u + GDEPTH) // U2) % 2
                    ej = (u + GDEPTH) % U2
                    bg = (u + GDEPTH) % RING

                    @pl.when(jj < K)
                    def _prefetch():
                        @pl.when(jj >= RING)
                        def _drain_prev():
                            drain_scatter(bg)
                        pltpu.async_copy(table_hbm.at[ebuf.at[pj, ej, 0]],
                                         rows[bg], gsems[bg])
            return carry

        lax.fori_loop(0, (K + UNROLL2 - 1) // UNROLL2, body, 0)
        for b in range(RING):
            drain_scatter(b)
        plsc.subcore_barrier()
        pltpu.sync_copy(acc.at[pl.ds(rbase, ROWS_PER_TILE)],
                        out_hbm.at[pl.ds(c * N + rbase, ROWS_PER_TILE)])

    return agg


_sc_agg_l1 = _make_sc_agg_l1()
_sc_agg_l2 = _make_sc_agg_l2()


ROW_BLK = 2000


def _tc1_body(aggp_ref, cntp_ref, x_ref, wl_ref, wr_ref, b_ref, h_ref):
    agg = aggp_ref[0] + aggp_ref[1]
    cnt = cntp_ref[0, :, 0:1] + cntp_ref[1, :, 0:1]
    mean = agg / jnp.maximum(cnt, 1.0)
    h = (jnp.dot(mean, wl_ref[...], preferred_element_type=jnp.float32)
         + jnp.dot(x_ref[...], wr_ref[...], preferred_element_type=jnp.float32)
         + b_ref[...])
    h = jnp.maximum(h, 0.0)
    h_ref[0] = h[:, :D_IN]
    h_ref[1] = h[:, D_IN:]


def _tc2_body(agg2_ref, cntp_ref, h2_ref, wl_ref, wr_ref, b_ref, o_ref):
    cnt = cntp_ref[0, :, 0:1] + cntp_ref[1, :, 0:1]
    inv = 1.0 / jnp.maximum(cnt, 1.0)
    o = (jnp.dot(agg2_ref[0] * inv, wl_ref[:D_IN, :],
                 preferred_element_type=jnp.float32)
         + jnp.dot(agg2_ref[1] * inv, wl_ref[D_IN:, :],
                   preferred_element_type=jnp.float32)
         + jnp.dot(h2_ref[0], wr_ref[:D_IN, :],
                   preferred_element_type=jnp.float32)
         + jnp.dot(h2_ref[1], wr_ref[D_IN:, :],
                   preferred_element_type=jnp.float32)
         + b_ref[...])
    o_ref[...] = o


def _tc1_call(agg1, cntp, x, W1_l, W1_r, b1):
    grid = N // ROW_BLK
    return pl.pallas_call(
        _tc1_body,
        grid=(grid,),
        in_specs=[
            pl.BlockSpec((NC, ROW_BLK, D_IN), lambda i: (0, i, 0)),
            pl.BlockSpec((NC, ROW_BLK, LANES), lambda i: (0, i, 0)),
            pl.BlockSpec((ROW_BLK, D_IN), lambda i: (i, 0)),
            pl.BlockSpec((D_IN, D_HID), lambda i: (0, 0)),
            pl.BlockSpec((D_IN, D_HID), lambda i: (0, 0)),
            pl.BlockSpec((1, D_HID), lambda i: (0, 0)),
        ],
        out_specs=pl.BlockSpec((NC, ROW_BLK, D_IN), lambda i: (0, i, 0)),
        out_shape=jax.ShapeDtypeStruct((NC, N, D_IN), jnp.float32),
    )(agg1, cntp, x, W1_l, W1_r, b1)


def _tc2_call(agg2, cntp, h2, W2_l, W2_r, b2):
    grid = N // ROW_BLK
    return pl.pallas_call(
        _tc2_body,
        grid=(grid,),
        in_specs=[
            pl.BlockSpec((NC, ROW_BLK, D_IN), lambda i: (0, i, 0)),
            pl.BlockSpec((NC, ROW_BLK, LANES), lambda i: (0, i, 0)),
            pl.BlockSpec((NC, ROW_BLK, D_IN), lambda i: (0, i, 0)),
            pl.BlockSpec((D_HID, D_HID), lambda i: (0, 0)),
            pl.BlockSpec((D_HID, D_HID), lambda i: (0, 0)),
            pl.BlockSpec((1, D_HID), lambda i: (0, 0)),
        ],
        out_specs=pl.BlockSpec((ROW_BLK, D_HID), lambda i: (i, 0)),
        out_shape=jax.ShapeDtypeStruct((N, D_HID), jnp.float32),
    )(agg2, cntp, h2, W2_l, W2_r, b2)


def kernel(x, edge_index, W1_l, W1_r, b1, W2_l, W2_r, b2):
    ei = edge_index.astype(jnp.int32)
    pad = PAD_CHUNKS * CHUNK - E
    src = jnp.pad(ei[0], (0, pad))
    src2d = src.reshape(PAD_CHUNKS, CHUNK)
    srcN2d = (src + N).reshape(PAD_CHUNKS, CHUNK)
    dst2d = jnp.pad(ei[1], (0, pad)).reshape(PAD_CHUNKS, CHUNK)
    e2 = jnp.stack([src2d, dst2d], axis=1)
    e2N = jnp.stack([srcN2d, dst2d], axis=1)
    z128 = jnp.zeros((ROWS_PER_TILE, D_IN), jnp.float32)
    z16 = jnp.zeros((ROWS_PER_TILE, LANES), jnp.float32)

    agg1, cnt1 = _sc_agg_l1(x, e2, z128, z16)
    agg1 = agg1.reshape(NC, N, D_IN)
    cntp = cnt1.reshape(NC, N, LANES)
    h2 = _tc1_call(agg1, cntp, x, W1_l, W1_r, b1.reshape(1, D_HID))
    agg2 = _sc_agg_l2(h2.reshape(NC * N, D_IN), e2, e2N, z128)
    agg2 = agg2.reshape(NC, N, D_IN)
    return _tc2_call(agg2, cntp, h2, W2_l, W2_r, b2.reshape(1, D_HID))


# speedup vs baseline: 1.0096x; 1.0096x over previous
import functools

import jax
import jax.numpy as jnp
from jax import lax
from jax.experimental import pallas as pl
from jax.experimental.pallas import tpu as pltpu
from jax.experimental.pallas import tpu_sc as plsc

N = 10000
E = 320000
D_IN = 128
D_HID = 256
NC = 2
NS = 16
LANES = 16
CHUNK = 80
ROWS_PER_TILE = N // NS
NCHUNKS_TOT = E // CHUNK
PAD_CHUNKS = 4002


RING1 = 3
GDEPTH1 = 2
U1 = 9
UNROLL1 = 2 * U1


def _make_sc_agg_l1():
    chunks_per_tile = NCHUNKS_TOT // (NC * NS)
    K = chunks_per_tile
    mesh = plsc.VectorSubcoreMesh(core_axis_name="c", subcore_axis_name="s")

    scratch = [pltpu.VMEM((2, U1, 2, CHUNK), jnp.int32)]
    scratch += [pltpu.VMEM((CHUNK, D_IN), jnp.float32) for _ in range(RING1)]
    scratch += [
        pltpu.VMEM((CHUNK, LANES), jnp.float32),
        pltpu.VMEM_SHARED((N, D_IN), jnp.float32),
        pltpu.VMEM_SHARED((N, LANES), jnp.float32),
    ]
    scratch += [pltpu.SemaphoreType.DMA for _ in range(3 * RING1 + 2)]

    @functools.partial(
        pl.kernel,
        out_type=(jax.ShapeDtypeStruct((NC * N, D_IN), jnp.float32),
                  jax.ShapeDtypeStruct((NC * N, LANES), jnp.float32)),
        mesh=mesh,
        scratch_types=scratch,
        compiler_params=pltpu.CompilerParams(use_tc_tiling_on_sc=False),
    )
    def agg(table_hbm, edges_hbm, z_hbm, z16_hbm, out_hbm, outc_hbm,
            ebuf, *rest):
        rows = list(rest[:RING1])
        ones = rest[RING1]
        acc = rest[RING1 + 1]
        accc = rest[RING1 + 2]
        gsems = list(rest[RING1 + 3:RING1 + 3 + RING1])
        ssems = list(rest[RING1 + 3 + RING1:RING1 + 3 + 2 * RING1])
        csems = list(rest[RING1 + 3 + 2 * RING1:RING1 + 3 + 3 * RING1])
        isems = list(rest[RING1 + 3 + 3 * RING1:])

        c = lax.axis_index("c")
        s = lax.axis_index("s")
        rbase = s * ROWS_PER_TILE
        cbase = (c * NS + s) * chunks_per_tile

        def load_group(gi, p):
            pltpu.async_copy(edges_hbm.at[pl.ds(cbase + gi * U1, U1)],
                             ebuf.at[p], isems[p])

        def wait_group(p):
            pltpu.make_async_copy(edges_hbm.at[pl.ds(0, U1)],
                                  ebuf.at[p], isems[p]).wait()

        def drain_gather(b):
            pltpu.make_async_copy(table_hbm.at[pl.ds(0, CHUNK)],
                                  rows[b], gsems[b]).wait()

        def drain_scatters(b):
            pltpu.make_async_copy(z_hbm.at[pl.ds(0, CHUNK)],
                                  rows[b], ssems[b]).wait()
            pltpu.make_async_copy(z16_hbm.at[pl.ds(0, CHUNK)],
                                  ones, csems[b]).wait()

        pltpu.sync_copy(edges_hbm.at[pl.ds(cbase, U1)], ebuf.at[0])
        for j0 in range(GDEPTH1):
            pltpu.async_copy(table_hbm.at[ebuf.at[0, j0, 0]],
                             rows[j0], gsems[j0])

        def fill(i, carry):
            ones[i] = jnp.full((LANES,), 1.0, jnp.float32)
            return carry
        lax.fori_loop(0, CHUNK, fill, 0)
        pltpu.sync_copy(z16_hbm.at[pl.ds(rbase, ROWS_PER_TILE)],
                        accc.at[pl.ds(rbase, ROWS_PER_TILE)])
        pltpu.sync_copy(z_hbm.at[pl.ds(rbase, ROWS_PER_TILE)],
                        acc.at[pl.ds(rbase, ROWS_PER_TILE)])
        plsc.subcore_barrier()

        def body(g, carry):
            base = g * UNROLL1
            for u in range(UNROLL1):
                j = base + u
                rs = u % RING1
                p = u // U1
                ei = u % U1

                if u == 1:
                    @pl.when((2 * g + 1) * U1 < K)
                    def _load_b():
                        load_group(2 * g + 1, 1)
                if u == U1 + 1:
                    @pl.when((2 * g + 2) * U1 < K)
                    def _load_a():
                        load_group(2 * g + 2, 0)
                if u == U1 - GDEPTH1:
                    @pl.when((2 * g + 1) * U1 < K)
                    def _wait_b():
                        wait_group(1)
                if u == UNROLL1 - GDEPTH1:
                    @pl.when((2 * g + 2) * U1 < K)
                    def _wait_a():
                        wait_group(0)

                @pl.when(j < K)
                def _process():
                    drain_gather(rs)
                    pltpu.async_copy(rows[rs], acc.at[ebuf.at[p, ei, 1]],
                                     ssems[rs], add=True)
                    pltpu.async_copy(ones, accc.at[ebuf.at[p, ei, 1]],
                                     csems[rs], add=True)
                    jj = j + GDEPTH1
                    pj = ((u + GDEPTH1) // U1) % 2
                    ej = (u + GDEPTH1) % U1
                    bg = (u + GDEPTH1) % RING1

                    @pl.when(jj < K)
                    def _prefetch():
                        @pl.when(jj >= RING1)
                        def _drain_prev():
                            drain_scatters(bg)
                        pltpu.async_copy(table_hbm.at[ebuf.at[pj, ej, 0]],
                                         rows[bg], gsems[bg])
            return carry

        lax.fori_loop(0, (K + UNROLL1 - 1) // UNROLL1, body, 0)
        for b in range(RING1):
            drain_scatters(b)
        plsc.subcore_barrier()
        pltpu.sync_copy(acc.at[pl.ds(rbase, ROWS_PER_TILE)],
                        out_hbm.at[pl.ds(c * N + rbase, ROWS_PER_TILE)])
        pltpu.sync_copy(accc.at[pl.ds(rbase, ROWS_PER_TILE)],
                        outc_hbm.at[pl.ds(c * N + rbase, ROWS_PER_TILE)])

    return agg


RING = 4
GDEPTH = 2
U2 = 12
UNROLL2 = 2 * U2


def _make_sc_agg_l2():
    chunks_per_tile = NCHUNKS_TOT // NS
    mesh = plsc.VectorSubcoreMesh(core_axis_name="c", subcore_axis_name="s")
    K = chunks_per_tile

    scratch = [pltpu.VMEM((2, U2, 2, CHUNK), jnp.int32)]
    scratch += [pltpu.VMEM((CHUNK, D_IN), jnp.float32) for _ in range(RING)]
    scratch += [pltpu.VMEM_SHARED((N, D_IN), jnp.float32)]
    scratch += [pltpu.SemaphoreType.DMA for _ in range(2 * RING + 2)]

    @functools.partial(
        pl.kernel,
        out_type=jax.ShapeDtypeStruct((NC * N, D_IN), jnp.float32),
        mesh=mesh,
        scratch_types=scratch,
        compiler_params=pltpu.CompilerParams(use_tc_tiling_on_sc=False),
    )
    def agg(table_hbm, edges_hbm, edgesN_hbm, z_hbm, out_hbm, ebuf, *rest):
        rows = list(rest[:RING])
        acc = rest[RING]
        gsems = list(rest[RING + 1:RING + 1 + RING])
        ssems = list(rest[RING + 1 + RING:RING + 1 + 2 * RING])
        isems = list(rest[RING + 1 + 2 * RING:])

        c = lax.axis_index("c")
        s = lax.axis_index("s")
        rbase = s * ROWS_PER_TILE
        cbase = s * chunks_per_tile

        def load_group(gi, p):
            @pl.when(c == 0)
            def _load0():
                pltpu.async_copy(edges_hbm.at[pl.ds(cbase + gi * U2, U2)],
                                 ebuf.at[p], isems[p])

            @pl.when(c == 1)
            def _load1():
                pltpu.async_copy(edgesN_hbm.at[pl.ds(cbase + gi * U2, U2)],
                                 ebuf.at[p], isems[p])

        def wait_group(p):
            pltpu.make_async_copy(edges_hbm.at[pl.ds(0, U2)],
                                  ebuf.at[p], isems[p]).wait()

        def drain_gather(b):
            pltpu.make_async_copy(table_hbm.at[pl.ds(0, CHUNK)],
                                  rows[b], gsems[b]).wait()

        def drain_scatter(b):
            pltpu.make_async_copy(z_hbm.at[pl.ds(0, CHUNK)],
                                  rows[b], ssems[b]).wait()

        load_group(0, 0)
        wait_group(0)
        for j0 in range(GDEPTH):
            pltpu.async_copy(table_hbm.at[ebuf.at[0, j0, 0]],
                             rows[j0], gsems[j0])
        pltpu.sync_copy(z_hbm.at[pl.ds(rbase, ROWS_PER_TILE)],
                        acc.at[pl.ds(rbase, ROWS_PER_TILE)])
        plsc.subcore_barrier()

        def body(g, carry):
            base = g * UNROLL2
            for u in range(UNROLL2):
                j = base + u
                rs = u % RING
                p = u // U2
                ei = u % U2

                if u == 2:
                    @pl.when((2 * g + 1) * U2 < K)
                    def _load_b():
                        load_group(2 * g + 1, 1)
                if u == U2 + 2:
                    @pl.when((2 * g + 2) * U2 < K)
                    def _load_a():
                        load_group(2 * g + 2, 0)
                if u == U2 - GDEPTH:
                    @pl.when((2 * g + 1) * U2 < K)
                    def _wait_b():
                        wait_group(1)
                if u == UNROLL2 - GDEPTH:
                    @pl.when((2 * g + 2) * U2 < K)
                    def _wait_a():
                        wait_group(0)

                @pl.when(j < K)
                def _process():
                    drain_gather(rs)
                    pltpu.async_copy(rows[rs], acc.at[ebuf.at[p, ei, 1]],
                                     ssems[rs], add=True)
                    jj = j + GDEPTH
                    pj = ((u + GDEPTH) // U2) % 2
                    ej = (u + GDEPTH) % U2
                    bg = (u + GDEPTH) % RING

                    @pl.when(jj < K)
                    def _prefetch():
                        @pl.when(jj >= RING)
                        def _drain_prev():
                            drain_scatter(bg)
                        pltpu.async_copy(table_hbm.at[ebuf.at[pj, ej, 0]],
                                         rows[bg], gsems[bg])
            return carry

        lax.fori_loop(0, (K + UNROLL2 - 1) // UNROLL2, body, 0)
        for b in range(RING):
            drain_scatter(b)
        plsc.subcore_barrier()
        pltpu.sync_copy(acc.at[pl.ds(rbase, ROWS_PER_TILE)],
                        out_hbm.at[pl.ds(c * N + rbase, ROWS_PER_TILE)])

    return agg


_sc_agg_l1 = _make_sc_agg_l1()
_sc_agg_l2 = _make_sc_agg_l2()


ROW_BLK = 2000


def _tc1_body(aggp_ref, cntp_ref, x_ref, wl_ref, wr_ref, b_ref, h_ref):
    agg = aggp_ref[0] + aggp_ref[1]
    cnt = cntp_ref[0, :, 0:1] + cntp_ref[1, :, 0:1]
    mean = agg / jnp.maximum(cnt, 1.0)
    h = (jnp.dot(mean, wl_ref[...], preferred_element_type=jnp.float32)
         + jnp.dot(x_ref[...], wr_ref[...], preferred_element_type=jnp.float32)
         + b_ref[...])
    h = jnp.maximum(h, 0.0)
    h_ref[0] = h[:, :D_IN]
    h_ref[1] = h[:, D_IN:]


def _tc2_body(agg2_ref, cntp_ref, h2_ref, wl_ref, wr_ref, b_ref, o_ref):
    cnt = cntp_ref[0, :, 0:1] + cntp_ref[1, :, 0:1]
    inv = 1.0 / jnp.maximum(cnt, 1.0)
    o = (jnp.dot(agg2_ref[0] * inv, wl_ref[:D_IN, :],
                 preferred_element_type=jnp.float32)
         + jnp.dot(agg2_ref[1] * inv, wl_ref[D_IN:, :],
                   preferred_element_type=jnp.float32)
         + jnp.dot(h2_ref[0], wr_ref[:D_IN, :],
                   preferred_element_type=jnp.float32)
         + jnp.dot(h2_ref[1], wr_ref[D_IN:, :],
                   preferred_element_type=jnp.float32)
         + b_ref[...])
    o_ref[...] = o


def _tc1_call(agg1, cntp, x, W1_l, W1_r, b1):
    grid = N // ROW_BLK
    return pl.pallas_call(
        _tc1_body,
        grid=(grid,),
        in_specs=[
            pl.BlockSpec((NC, ROW_BLK, D_IN), lambda i: (0, i, 0)),
            pl.BlockSpec((NC, ROW_BLK, LANES), lambda i: (0, i, 0)),
            pl.BlockSpec((ROW_BLK, D_IN), lambda i: (i, 0)),
            pl.BlockSpec((D_IN, D_HID), lambda i: (0, 0)),
            pl.BlockSpec((D_IN, D_HID), lambda i: (0, 0)),
            pl.BlockSpec((1, D_HID), lambda i: (0, 0)),
        ],
        out_specs=pl.BlockSpec((NC, ROW_BLK, D_IN), lambda i: (0, i, 0)),
        out_shape=jax.ShapeDtypeStruct((NC, N, D_IN), jnp.float32),
    )(agg1, cntp, x, W1_l, W1_r, b1)


def _tc2_call(agg2, cntp, h2, W2_l, W2_r, b2):
    grid = N // ROW_BLK
    return pl.pallas_call(
        _tc2_body,
        grid=(grid,),
        in_specs=[
            pl.BlockSpec((NC, ROW_BLK, D_IN), lambda i: (0, i, 0)),
            pl.BlockSpec((NC, ROW_BLK, LANES), lambda i: (0, i, 0)),
            pl.BlockSpec((NC, ROW_BLK, D_IN), lambda i: (0, i, 0)),
            pl.BlockSpec((D_HID, D_HID), lambda i: (0, 0)),
            pl.BlockSpec((D_HID, D_HID), lambda i: (0, 0)),
            pl.BlockSpec((1, D_HID), lambda i: (0, 0)),
        ],
        out_specs=pl.BlockSpec((ROW_BLK, D_HID), lambda i: (i, 0)),
        out_shape=jax.ShapeDtypeStruct((N, D_HID), jnp.float32),
    )(agg2, cntp, h2, W2_l, W2_r, b2)


def kernel(x, edge_index, W1_l, W1_r, b1, W2_l, W2_r, b2):
    ei = edge_index.astype(jnp.int32)
    pad = PAD_CHUNKS * CHUNK - E
    src = jnp.pad(ei[0], (0, pad))
    src2d = src.reshape(PAD_CHUNKS, CHUNK)
    srcN2d = (src + N).reshape(PAD_CHUNKS, CHUNK)
    dst2d = jnp.pad(ei[1], (0, pad)).reshape(PAD_CHUNKS, CHUNK)
    e2 = jnp.stack([src2d, dst2d], axis=1)
    e2N = jnp.stack([srcN2d, dst2d], axis=1)
    z128 = jnp.zeros((N, D_IN), jnp.float32)
    z16 = jnp.zeros((N, LANES), jnp.float32)

    agg1, cnt1 = _sc_agg_l1(x, e2, z128, z16)
    agg1 = agg1.reshape(NC, N, D_IN)
    cntp = cnt1.reshape(NC, N, LANES)
    h2 = _tc1_call(agg1, cntp, x, W1_l, W1_r, b1.reshape(1, D_HID))
    agg2 = _sc_agg_l2(h2.reshape(NC * N, D_IN), e2, e2N, z128)
    agg2 = agg2.reshape(NC, N, D_IN)
    return _tc2_call(agg2, cntp, h2, W2_l, W2_r, b2.reshape(1, D_HID))
